# baseline (speedup 1.0000x reference)
import functools

import jax
import jax.numpy as jnp
from jax import lax
from jax.experimental import pallas as pl
from jax.experimental.pallas import tpu as pltpu
from jax.experimental.pallas import tpu_sc as plsc

_B = 16
_T = 4096
_CH = 16
_NCHUNKS = _T // _CH
_NW = 32
_TC_RES = 6
_SC_UNITS = 8 - _TC_RES
_AW = 16 * _TC_RES
_NB = _T // 128


def _sc_counts(x_flat, dia_lens):
    mesh = plsc.VectorSubcoreMesh(core_axis_name="c", subcore_axis_name="s")

    @functools.partial(
        pl.kernel,
        mesh=mesh,
        out_type=jax.ShapeDtypeStruct((_NW, _B * _CH), jnp.int32),
        scratch_types=[
            pltpu.VMEM((_B * _T,), jnp.float32),
            pltpu.VMEM((_B,), jnp.int32),
            pltpu.VMEM((_B * _CH,), jnp.int32),
            pltpu.SemaphoreType.DMA,
        ],
    )
    def k(x_hbm, lens_hbm, out_hbm, x_v, lens_v, acc_v, sem):
        wid = lax.axis_index("s") * 2 + lax.axis_index("c")
        pltpu.sync_copy(lens_hbm, lens_v)
        handles = [
            pltpu.async_copy(x_hbm.at[pl.ds(row * _T, _T)],
                             x_v.at[pl.ds(row * _T, _T)], sem)
            for row in range(_B)
        ]
        iota = lax.iota(jnp.int32, _CH)
        neg_inf = jnp.float32(-jnp.inf)
        zeros = jnp.zeros((_CH,), jnp.int32)
        lreg = lens_v[...]

        for row in range(_B):
            handles[row].wait()
            L = lreg[row]
            Lv = jnp.full((_CH,), L, jnp.int32)
            ceil_chunks = (L + _CH - 1) // _CH

            def unit_body(jj, tot, row=row, L=L, Lv=Lv,
                          ceil_chunks=ceil_chunks):
                if row % 2 == 1:
                    c = 8 * (_NW - 1 - wid) + _TC_RES + jj
                else:
                    c = 8 * wid + _TC_RES + jj
                base = c * _CH
                jmax = jnp.minimum(c, ceil_chunks)

                va = x_v[pl.ds(row * _T + base, _CH)]
                va_m = jnp.where(base + iota < Lv, va, neg_inf)
                bs = [jnp.full((_CH,), va_m[i], jnp.float32)
                      for i in range(_CH)]
                accs = [zeros for _ in range(_CH)]
                for i in range(1, _CH):
                    m = (iota < i) & (va_m < bs[i])
                    accs[i] = accs[i] + jnp.where(m, 1, 0)

                def jloop(j, acc_t):
                    vb = x_v[pl.ds(row * _T + j * _CH, _CH)]
                    return tuple(a + jnp.where(vb < b, 1, 0)
                                 for a, b in zip(acc_t, bs))

                accs2 = list(
                    plsc.parallel_loop(0, jmax, 1, unroll=4,
                                       carry=tuple(accs))(jloop))
                while len(accs2) > 1:
                    nxt = [accs2[2 * i] + accs2[2 * i + 1]
                           for i in range(len(accs2) // 2)]
                    if len(accs2) % 2:
                        nxt.append(accs2[-1])
                    accs2 = nxt
                return tot + accs2[0]

            tot = lax.fori_loop(0, _SC_UNITS, unit_body, zeros)
            acc_v[pl.ds(row * _CH, _CH)] = tot

        pltpu.sync_copy(acc_v, out_hbm.at[wid])

    return k(x_flat, dia_lens)


def _tc_count_and_loss(x, xt, lens_col):
    def body(x_ref, xt_ref, lc_ref, loss_ref, cnt_ref):
        xv = x_ref[...]
        lens = lc_ref[...]
        pos = lax.broadcasted_iota(jnp.int32, (_B, _T), 1)
        mask = pos >= lens
        lf = lens.astype(jnp.float32)
        lin = pos.astype(jnp.float32) / (lf - 1.0)
        padded = jnp.where(mask, jnp.float32(1.0), lin)
        q = 2.0 * padded
        q2 = q * q
        q5 = q2 * q2 * q
        rt = 1.0 / (1.0 + q5)
        ml = jnp.where(mask, -jnp.inf, xv)
        kl = rt * (jnp.log(rt) - ml)
        loss_ref[0, 0] = jnp.sum(kl) / jnp.float32(_B)

        neg_inf = jnp.float32(-jnp.inf)
        lane_iota = lax.broadcasted_iota(jnp.int32, (1, 128), 1)
        sub_iota = lax.broadcasted_iota(jnp.int32, (_AW, 1), 0)
        for r in range(_B):
            Lr = lens[r, 0]
            jb_ceil = (Lr + 127) // 128

            def band_body(m, acc, r=r, Lr=Lr, jb_ceil=jb_ceil):
                a_pos = m * 128 + sub_iota
                xa = xt_ref[pl.ds(m * 128, _AW), pl.ds(r, 1)]
                A = jnp.where(a_pos < Lr, xa, neg_inf)
                Abc = A + jnp.zeros((_AW, 128), jnp.float32)

                nfull = jnp.minimum(m, jb_ceil)

                def jb_body(i, acc2, r=r, Abc=Abc):
                    for kk in range(4):
                        Bk = x_ref[pl.ds(r, 1), pl.ds(i * 512 + kk * 128, 128)]
                        acc2 = acc2 + jnp.where(Bk < Abc, 1, 0)
                    return acc2

                acc = lax.fori_loop(0, nfull // 4, jb_body, acc)
                for t in range(3):
                    jt = jnp.maximum(nfull - 1 - t, 0)
                    Bt = x_ref[pl.ds(r, 1), pl.ds(jt * 128, 128)]
                    acc = acc + jnp.where(((nfull & 3) > t) & (Bt < Abc), 1, 0)
                Bv = x_ref[pl.ds(r, 1), pl.ds(m * 128, 128)]
                apos_bc = a_pos + jnp.zeros((_AW, 128), jnp.int32)
                b_pos = m * 128 + lane_iota
                md = (b_pos < apos_bc) & (Bv < Abc)
                return acc + jnp.where(md, 1, 0)

            acc = lax.fori_loop(
                0, _NB, band_body, jnp.zeros((_AW, 128), jnp.int32))
            cnt_ref[r, 0] = jnp.sum(acc).astype(jnp.float32)

    loss, cnt = pl.pallas_call(
        body,
        out_shape=[
            jax.ShapeDtypeStruct((1, 1), jnp.float32),
            jax.ShapeDtypeStruct((_B, 1), jnp.float32),
        ],
        out_specs=[
            pl.BlockSpec(memory_space=pltpu.SMEM),
            pl.BlockSpec(memory_space=pltpu.SMEM),
        ],
    )(x, xt, lens_col)
    return loss, cnt


def _tc_finalize(sc_counts, tc_cnt, lens_col):
    def body(sc_ref, tc_ref, lc_ref, si_ref):
        inv_sc = jnp.sum(sc_ref[...].astype(jnp.float32), axis=1,
                         keepdims=True)
        total = inv_sc + tc_ref[...]
        lf = lc_ref[...].astype(jnp.float32)
        max_inv = lf * (lf - 1.0) * 0.5
        si_ref[0, 0] = 1.0 - jnp.sum(total / max_inv) / jnp.float32(_B)

    si = pl.pallas_call(
        body,
        out_shape=jax.ShapeDtypeStruct((1, 1), jnp.float32),
        out_specs=pl.BlockSpec(memory_space=pltpu.SMEM),
    )(sc_counts, tc_cnt, lens_col)
    return si


def kernel(ranks_logits, dia_lens):
    lens_col = dia_lens.reshape(_B, 1)
    sc_counts = _sc_counts(ranks_logits.reshape(-1), dia_lens)
    loss, tc_cnt = _tc_count_and_loss(
        ranks_logits, ranks_logits.T, lens_col)
    sc_counts = (
        sc_counts.reshape(_NW, _B, _CH).transpose(1, 0, 2).reshape(_B, -1))
    si = _tc_finalize(sc_counts, tc_cnt, lens_col)
    return (loss[0, 0], si[0, 0])


# speedup vs baseline: 1.1703x; 1.1703x over previous
import functools

import jax
import jax.numpy as jnp
from jax import lax
from jax.experimental import pallas as pl
from jax.experimental.pallas import tpu as pltpu
from jax.experimental.pallas import tpu_sc as plsc

_B = 16
_T = 4096
_CH = 16
_NCHUNKS = _T // _CH
_NW = 32
_TC_RES = 4
_SC_UNITS = 8 - _TC_RES
_AW = 16 * _TC_RES
_NB = _T // 128


def _sc_counts(x_flat, dia_lens):
    mesh = plsc.VectorSubcoreMesh(core_axis_name="c", subcore_axis_name="s")

    @functools.partial(
        pl.kernel,
        mesh=mesh,
        out_type=jax.ShapeDtypeStruct((_NW, _B * _CH), jnp.int32),
        scratch_types=[
            pltpu.VMEM((_B * _T,), jnp.float32),
            pltpu.VMEM((_B,), jnp.int32),
            pltpu.VMEM((_B * _CH,), jnp.int32),
            pltpu.SemaphoreType.DMA,
        ],
    )
    def k(x_hbm, lens_hbm, out_hbm, x_v, lens_v, acc_v, sem):
        wid = lax.axis_index("s") * 2 + lax.axis_index("c")
        pltpu.sync_copy(lens_hbm, lens_v)
        handles = [
            pltpu.async_copy(x_hbm.at[pl.ds(row * _T, _T)],
                             x_v.at[pl.ds(row * _T, _T)], sem)
            for row in range(_B)
        ]
        iota = lax.iota(jnp.int32, _CH)
        neg_inf = jnp.float32(-jnp.inf)
        zeros = jnp.zeros((_CH,), jnp.int32)
        lreg = lens_v[...]

        for row in range(_B):
            handles[row].wait()
            L = lreg[row]
            Lv = jnp.full((_CH,), L, jnp.int32)
            ceil_chunks = (L + _CH - 1) // _CH

            def unit_body(jj, tot, row=row, L=L, Lv=Lv,
                          ceil_chunks=ceil_chunks):
                if row % 2 == 1:
                    c = 8 * (_NW - 1 - wid) + _TC_RES + jj
                else:
                    c = 8 * wid + _TC_RES + jj
                base = c * _CH
                jmax = jnp.minimum(c, ceil_chunks)

                va = x_v[pl.ds(row * _T + base, _CH)]
                va_m = jnp.where(base + iota < Lv, va, neg_inf)
                bs = [jnp.full((_CH,), va_m[i], jnp.float32)
                      for i in range(_CH)]
                accs = [zeros for _ in range(_CH)]
                for i in range(1, _CH):
                    m = (iota < i) & (va_m < bs[i])
                    accs[i] = accs[i] + jnp.where(m, 1, 0)

                def jloop(j, acc_t):
                    vb = x_v[pl.ds(row * _T + j * _CH, _CH)]
                    return tuple(a + jnp.where(vb < b, 1, 0)
                                 for a, b in zip(acc_t, bs))

                accs2 = list(
                    plsc.parallel_loop(0, jmax, 1, unroll=4,
                                       carry=tuple(accs))(jloop))
                while len(accs2) > 1:
                    nxt = [accs2[2 * i] + accs2[2 * i + 1]
                           for i in range(len(accs2) // 2)]
                    if len(accs2) % 2:
                        nxt.append(accs2[-1])
                    accs2 = nxt
                return tot + accs2[0]

            tot = lax.fori_loop(0, _SC_UNITS, unit_body, zeros)
            acc_v[pl.ds(row * _CH, _CH)] = tot

        pltpu.sync_copy(acc_v, out_hbm.at[wid])

    return k(x_flat, dia_lens)


def _tc_count_and_loss(x, xt, lens_col):
    def body(x_ref, xt_ref, lc_ref, loss_ref, cnt_ref):
        xv = x_ref[...]
        lens = lc_ref[...]
        pos = lax.broadcasted_iota(jnp.int32, (_B, _T), 1)
        mask = pos >= lens
        lf = lens.astype(jnp.float32)
        lin = pos.astype(jnp.float32) / (lf - 1.0)
        padded = jnp.where(mask, jnp.float32(1.0), lin)
        q = 2.0 * padded
        q2 = q * q
        q5 = q2 * q2 * q
        rt = 1.0 / (1.0 + q5)
        ml = jnp.where(mask, -jnp.inf, xv)
        kl = rt * (jnp.log(rt) - ml)
        loss_ref[0, 0] = jnp.sum(kl) / jnp.float32(_B)

        neg_inf = jnp.float32(-jnp.inf)
        lane_iota = lax.broadcasted_iota(jnp.int32, (1, 128), 1)
        sub_iota = lax.broadcasted_iota(jnp.int32, (_AW, 1), 0)
        for r in range(_B):
            Lr = lens[r, 0]
            jb_ceil = (Lr + 127) // 128

            def band_body(m, acc, r=r, Lr=Lr, jb_ceil=jb_ceil):
                a_pos = m * 128 + sub_iota
                xa = xt_ref[pl.ds(m * 128, _AW), pl.ds(r, 1)]
                A = jnp.where(a_pos < Lr, xa, neg_inf)
                Abc = A + jnp.zeros((_AW, 128), jnp.float32)

                nfull = jnp.minimum(m, jb_ceil)

                def jb_body(i, acc2, r=r, Abc=Abc):
                    for kk in range(4):
                        Bk = x_ref[pl.ds(r, 1), pl.ds(i * 512 + kk * 128, 128)]
                        acc2 = acc2 + jnp.where(Bk < Abc, 1, 0)
                    return acc2

                acc = lax.fori_loop(0, nfull // 4, jb_body, acc)
                for t in range(3):
                    jt = jnp.maximum(nfull - 1 - t, 0)
                    Bt = x_ref[pl.ds(r, 1), pl.ds(jt * 128, 128)]
                    acc = acc + jnp.where(((nfull & 3) > t) & (Bt < Abc), 1, 0)
                Bv = x_ref[pl.ds(r, 1), pl.ds(m * 128, 128)]
                apos_bc = a_pos + jnp.zeros((_AW, 128), jnp.int32)
                b_pos = m * 128 + lane_iota
                md = (b_pos < apos_bc) & (Bv < Abc)
                return acc + jnp.where(md, 1, 0)

            acc = lax.fori_loop(
                0, _NB, band_body, jnp.zeros((_AW, 128), jnp.int32))
            cnt_ref[0, r] = jnp.sum(acc).astype(jnp.float32)

    loss, cnt = pl.pallas_call(
        body,
        out_shape=[
            jax.ShapeDtypeStruct((1, 1), jnp.float32),
            jax.ShapeDtypeStruct((1, _B), jnp.float32),
        ],
        out_specs=[
            pl.BlockSpec(memory_space=pltpu.SMEM),
            pl.BlockSpec(memory_space=pltpu.SMEM),
        ],
    )(x, xt, lens_col)
    return loss, cnt


def _tc_finalize(sc_counts, tc_cnt, lens_row):
    def body(sc_ref, tc_ref, lr_ref, si_ref):
        c = sc_ref[...].astype(jnp.float32)
        src = lax.broadcasted_iota(jnp.int32, (_B * _CH, _B), 0) // _CH
        dst = lax.broadcasted_iota(jnp.int32, (_B * _CH, _B), 1)
        onehot = jnp.where(src == dst, 1.0, 0.0)
        inv_sc = jnp.sum(jnp.dot(c, onehot,
                                 preferred_element_type=jnp.float32),
                         axis=0, keepdims=True)
        total = inv_sc + tc_ref[...]
        lf = lr_ref[...].astype(jnp.float32)
        max_inv = lf * (lf - 1.0) * 0.5
        si_ref[0, 0] = 1.0 - jnp.sum(total / max_inv) / jnp.float32(_B)

    si = pl.pallas_call(
        body,
        out_shape=jax.ShapeDtypeStruct((1, 1), jnp.float32),
        out_specs=pl.BlockSpec(memory_space=pltpu.SMEM),
    )(sc_counts, tc_cnt, lens_row)
    return si


def kernel(ranks_logits, dia_lens):
    lens_col = dia_lens.reshape(_B, 1)
    sc_counts = _sc_counts(ranks_logits.reshape(-1), dia_lens)
    loss, tc_cnt = _tc_count_and_loss(
        ranks_logits, ranks_logits.T, lens_col)
    si = _tc_finalize(sc_counts, tc_cnt, dia_lens.reshape(1, _B))
    return (loss[0, 0], si[0, 0])
